# baseline (speedup 1.0000x reference)
import jax
import jax.numpy as jnp
from jax import lax
from jax.experimental import pallas as pl
from jax.experimental.pallas import tpu as pltpu
from jax.experimental.pallas import tpu_sc as plsc

PREFIX_SIZE = 1024
KV_SIZE = 16384
BATCH = 16
PREFIX_LEN = 128
B = BATCH * PREFIX_LEN
NC, NS = 2, 16
NW = NC * NS
BPW = B // NW

F = 8
DV = KV_SIZE // F
VF = PREFIX_SIZE * F
BF = B * F
SLEN = 8
NCH = BPW * F // SLEN
NBUF = 6
M = ((NCH - NBUF) // NBUF) * NBUF

assert NBUF * SLEN * DV + BPW * F <= 131071


def _body(table_hbm, idx_hbm, out_hbm, idx_v, bufs, in_sems, out_sems):
    wid = lax.axis_index("s") * NC + lax.axis_index("c")
    fbase = wid * BPW * F

    pltpu.sync_copy(idx_hbm.at[pl.ds(fbase, BPW * F)], idx_v)

    def start_gather(cg, b):
        pltpu.async_copy(
            table_hbm.at[idx_v.at[pl.ds(cg * SLEN, SLEN)]],
            bufs.at[b],
            in_sems.at[b],
        )

    def wait_gather(b):
        pltpu.make_async_copy(
            table_hbm.at[pl.ds(0, SLEN)], bufs.at[b], in_sems.at[b]
        ).wait()

    def start_out(cg, b):
        pltpu.async_copy(
            bufs.at[b], out_hbm.at[pl.ds(fbase + cg * SLEN, SLEN)],
            out_sems.at[b],
        )

    def wait_out(b):
        pltpu.make_async_copy(
            bufs.at[b], out_hbm.at[pl.ds(0, SLEN)], out_sems.at[b]
        ).wait()

    for b in range(NBUF):
        start_gather(b, b)

    @pl.loop(0, M, step=NBUF)
    def _(g0):
        for b in range(NBUF):
            cg = g0 + b
            wait_gather(b)
            start_out(cg, b)
            wait_out(b)
            start_gather(cg + NBUF, b)

    for cg in range(M, NCH):
        b = cg % NBUF
        wait_gather(b)
        start_out(cg, b)
        if cg + NBUF < NCH:
            wait_out(b)
            start_gather(cg + NBUF, b)
    for cg in range(max(M, NCH - NBUF), NCH):
        wait_out(cg % NBUF)


@jax.jit
def _gather(table_f, idx_flat):
    mesh = plsc.VectorSubcoreMesh(
        core_axis_name="c", subcore_axis_name="s", num_cores=NC, num_subcores=NS
    )
    f = pl.kernel(
        _body,
        out_type=jax.ShapeDtypeStruct((BF, DV), jnp.float32),
        mesh=mesh,
        scratch_types=[
            pltpu.VMEM((BPW * F,), jnp.int32),
            pltpu.VMEM((NBUF, SLEN, DV), jnp.float32),
            pltpu.SemaphoreType.DMA((NBUF,)),
            pltpu.SemaphoreType.DMA((NBUF,)),
        ],
    )
    return f(table_f, idx_flat)


def kernel(prefix, table):
    idx = prefix.reshape(B)
    idx_flat = (idx[:, None] * F + jnp.arange(F, dtype=jnp.int32)).reshape(BF)
    table_f = table.reshape(VF, DV)
    out = _gather(table_f, idx_flat)
    return out.reshape(BATCH, PREFIX_LEN, KV_SIZE)


# speedup vs baseline: 3.0333x; 3.0333x over previous
import jax, jax.numpy as jnp
from jax import lax
from jax.experimental import pallas as pl
from jax.experimental.pallas import tpu as pltpu
from jax.experimental.pallas import tpu_sc as plsc

B, D = 2048, 16384
BPW = 64
NBUF = 7
NC, NS = 2, 16


def _body(t4, idx_hbm, out4, idx_v, *rest):
    bufs = rest[:NBUF]
    in_sems, out_sems = rest[NBUF], rest[NBUF + 1]
    wid = lax.axis_index("s") * NC + lax.axis_index("c")
    base = wid * BPW

    pltpu.sync_copy(idx_hbm.at[pl.ds(base, BPW)], idx_v)
    rows = []
    for blk in range(BPW // 16):
        v = idx_v[pl.ds(blk * 16, 16)]
        rows.extend(v[j] for j in range(16))

    def start_gather(g, b):
        r = rows[g]
        pltpu.async_copy(
            t4.at[pl.ds(r // 8, 1), :, pl.ds(r % 8, 1), :], bufs[b],
            in_sems.at[b],
        )

    def wait_gather(b):
        pltpu.make_async_copy(
            t4.at[pl.ds(0, 1), :, pl.ds(0, 1), :], bufs[b], in_sems.at[b]
        ).wait()

    def start_out(g, b):
        pltpu.async_copy(
            bufs[b],
            out4.at[pl.ds(wid * 8 + g // 8, 1), :, pl.ds(g % 8, 1), :],
            out_sems.at[b],
        )

    for b in range(NBUF):
        start_gather(b, b)
    for g in range(BPW):
        b = g % NBUF
        wait_gather(b)
        out_dma = start_out(g, b)
        pltpu.make_async_copy(
            bufs[b], t4.at[pl.ds(0, 1), :, pl.ds(0, 1), :], out_sems.at[b]
        ).wait()
        if g + NBUF < BPW:
            start_gather(g + NBUF, b)


def kernel(prefix, table):
    idx = prefix.reshape(B)
    t4 = table.reshape(128, 8, 128, 128).transpose(0, 2, 1, 3)
    mesh = plsc.VectorSubcoreMesh(core_axis_name="c", subcore_axis_name="s",
                                  num_cores=NC, num_subcores=NS)
    f = pl.kernel(
        _body,
        out_type=jax.ShapeDtypeStruct((B // 8, 128, 8, 128), jnp.float32),
        mesh=mesh,
        scratch_types=(
            [pltpu.VMEM((BPW,), jnp.int32)]
            + [pltpu.VMEM((1, 128, 1, 128), jnp.float32) for _ in range(NBUF)]
            + [pltpu.SemaphoreType.DMA((NBUF,)),
               pltpu.SemaphoreType.DMA((NBUF,))]
        ),
    )
    out4 = f(t4, idx)
    return out4.transpose(0, 2, 1, 3).reshape(16, 128, D)


# speedup vs baseline: 3.0372x; 1.0013x over previous
import jax
import jax.numpy as jnp
from jax import lax
from jax.experimental import pallas as pl
from jax.experimental.pallas import tpu as pltpu
from jax.experimental.pallas import tpu_sc as plsc

B, D = 2048, 16384
BPW = 64
NBUF = 7
LA = 4
NC, NS = 2, 16


def _body(t4, idx_hbm, out4, idx_v, *rest):
    bufs = rest[:NBUF]
    in_sems, out_sems = rest[NBUF], rest[NBUF + 1]
    wid = lax.axis_index("s") * NC + lax.axis_index("c")
    base = wid * BPW

    pltpu.sync_copy(idx_hbm.at[pl.ds(base, BPW)], idx_v)
    rows = []
    for blk in range(BPW // 16):
        v = idx_v[pl.ds(blk * 16, 16)]
        rows.extend(v[j] for j in range(16))

    def start_gather(g, b):
        r = rows[g]
        pltpu.async_copy(
            t4.at[pl.ds(r // 8, 1), :, pl.ds(r % 8, 1), :], bufs[b],
            in_sems.at[b],
        )

    def wait_gather(b):
        pltpu.make_async_copy(
            t4.at[pl.ds(0, 1), :, pl.ds(0, 1), :], bufs[b], in_sems.at[b]
        ).wait()

    def start_out(g, b):
        pltpu.async_copy(
            bufs[b],
            out4.at[pl.ds(wid * 8 + g // 8, 1), :, pl.ds(g % 8, 1), :],
            out_sems.at[b],
        )

    def wait_out(b):
        pltpu.make_async_copy(
            bufs[b], t4.at[pl.ds(0, 1), :, pl.ds(0, 1), :], out_sems.at[b]
        ).wait()

    for h in range(LA):
        start_gather(h, h % NBUF)
    for g in range(BPW):
        b = g % NBUF
        wait_gather(b)
        start_out(g, b)
        h = g + LA
        if h < BPW:
            bh = h % NBUF
            if h >= NBUF:
                wait_out(bh)
            start_gather(h, bh)
    for g in range(BPW - NBUF, BPW):
        wait_out(g % NBUF)


def kernel(prefix, table):
    idx = prefix.reshape(B)
    t4 = table.reshape(128, 8, 128, 128).transpose(0, 2, 1, 3)
    mesh = plsc.VectorSubcoreMesh(core_axis_name="c", subcore_axis_name="s",
                                  num_cores=NC, num_subcores=NS)
    f = pl.kernel(
        _body,
        out_type=jax.ShapeDtypeStruct((B // 8, 128, 8, 128), jnp.float32),
        mesh=mesh,
        scratch_types=(
            [pltpu.VMEM((BPW,), jnp.int32)]
            + [pltpu.VMEM((1, 128, 1, 128), jnp.float32) for _ in range(NBUF)]
            + [pltpu.SemaphoreType.DMA((NBUF,)),
               pltpu.SemaphoreType.DMA((NBUF,))]
        ),
    )
    out4 = f(t4, idx)
    return out4.transpose(0, 2, 1, 3).reshape(16, 128, D)
